# baseline (speedup 1.0000x reference)
import jax
import jax.numpy as jnp
from jax.experimental import pallas as pl
from jax.experimental.pallas import tpu as pltpu

_TAU_NUM_STEPS = 50
_NUM_TRAIN_STEPS = 1000


def _tau_tables():
    betas = jnp.linspace(1e-4, 0.02, _NUM_TRAIN_STEPS, dtype=jnp.float32)
    alphas = 1.0 - betas
    alpha_bars = jnp.cumprod(alphas)
    tau = jnp.linspace(0, _NUM_TRAIN_STEPS - 1, _TAU_NUM_STEPS).astype(jnp.int32)
    tau_alpha_bars = jnp.take(alpha_bars, tau)
    sac = jnp.sqrt(tau_alpha_bars)
    somac = jnp.sqrt(1.0 - tau_alpha_bars)
    return sac, somac


def _body(ts_ref, pts_ref, sac_ref, somac_ref, xt_ref, pn_ref, xtp_ref, x0_ref):
    i = pl.program_id(0)
    t = ts_ref[i]
    p = pts_ref[i]
    sac = sac_ref[t]
    somac = somac_ref[t]
    psac = sac_ref[p]
    psomac = somac_ref[p]
    dc = jnp.sqrt(jnp.maximum(psomac * psomac, jnp.float32(1e-8)))
    rsac = 1.0 / sac
    c = rsac
    d = -rsac * somac
    a = psac * c
    b = psac * d + dc
    xt_v = xt_ref[...]
    pn = pn_ref[...]
    x0_ref[...] = c * xt_v + d * pn
    xtp_ref[...] = a * xt_v + b * pn


_SPLIT = 2


def kernel(xt, predicted_noise, time_steps, prev_time_steps):
    B, C, H, W = xt.shape
    rows = C * H * W // W
    rb = rows // _SPLIT
    sac, somac = _tau_tables()
    x3 = xt.reshape(B, rows, W)
    p3 = predicted_noise.reshape(B, rows, W)
    grid_spec = pltpu.PrefetchScalarGridSpec(
        num_scalar_prefetch=4,
        grid=(B, _SPLIT),
        in_specs=[
            pl.BlockSpec((1, rb, W), lambda i, j, *_: (i, j, 0)),
            pl.BlockSpec((1, rb, W), lambda i, j, *_: (i, j, 0)),
        ],
        out_specs=[
            pl.BlockSpec((1, rb, W), lambda i, j, *_: (i, j, 0)),
            pl.BlockSpec((1, rb, W), lambda i, j, *_: (i, j, 0)),
        ],
    )
    xtp, x0 = pl.pallas_call(
        _body,
        grid_spec=grid_spec,
        out_shape=[jax.ShapeDtypeStruct((B, rows, W), jnp.float32)] * 2,
        compiler_params=pltpu.CompilerParams(
            dimension_semantics=("parallel", "parallel"),
        ),
    )(
        time_steps.astype(jnp.int32),
        prev_time_steps.astype(jnp.int32),
        sac,
        somac,
        x3,
        p3,
    )
    return xtp.reshape(B, C, H, W), x0.reshape(B, C, H, W)


# speedup vs baseline: 1.7635x; 1.7635x over previous
import jax
import jax.numpy as jnp
from jax.experimental import pallas as pl
from jax.experimental.pallas import tpu as pltpu

_TAU_NUM_STEPS = 50
_NUM_TRAIN_STEPS = 1000


def _tau_tables():
    betas = jnp.linspace(1e-4, 0.02, _NUM_TRAIN_STEPS, dtype=jnp.float32)
    alphas = 1.0 - betas
    alpha_bars = jnp.cumprod(alphas)
    tau = jnp.linspace(0, _NUM_TRAIN_STEPS - 1, _TAU_NUM_STEPS).astype(jnp.int32)
    tau_alpha_bars = jnp.take(alpha_bars, tau)
    sac = jnp.sqrt(tau_alpha_bars)
    somac = jnp.sqrt(1.0 - tau_alpha_bars)
    return sac, somac


def _body(ts_ref, pts_ref, sac_ref, somac_ref, xt_ref, pn_ref, xtp_ref, x0_ref):
    i = pl.program_id(0)
    g = xt_ref.shape[0]
    for k in range(g):
        s = i * g + k
        t = ts_ref[s]
        p = pts_ref[s]
        sac = sac_ref[t]
        somac = somac_ref[t]
        psac = sac_ref[p]
        psomac = somac_ref[p]
        dc = jnp.sqrt(jnp.maximum(psomac * psomac, jnp.float32(1e-8)))
        rsac = 1.0 / sac
        c = rsac
        d = -rsac * somac
        a = psac * c
        b = psac * d + dc
        xt_v = xt_ref[k]
        pn = pn_ref[k]
        x0_ref[k] = c * xt_v + d * pn
        xtp_ref[k] = a * xt_v + b * pn


_GROUP = 2


def kernel(xt, predicted_noise, time_steps, prev_time_steps):
    B, C, H, W = xt.shape
    rows = C * H * W // W
    sac, somac = _tau_tables()
    x3 = xt.reshape(B, rows, W)
    p3 = predicted_noise.reshape(B, rows, W)
    g = _GROUP
    grid_spec = pltpu.PrefetchScalarGridSpec(
        num_scalar_prefetch=4,
        grid=(B // g,),
        in_specs=[
            pl.BlockSpec((g, rows, W), lambda i, *_: (i, 0, 0)),
            pl.BlockSpec((g, rows, W), lambda i, *_: (i, 0, 0)),
        ],
        out_specs=[
            pl.BlockSpec((g, rows, W), lambda i, *_: (i, 0, 0)),
            pl.BlockSpec((g, rows, W), lambda i, *_: (i, 0, 0)),
        ],
    )
    xtp, x0 = pl.pallas_call(
        _body,
        grid_spec=grid_spec,
        out_shape=[jax.ShapeDtypeStruct((B, rows, W), jnp.float32)] * 2,
        compiler_params=pltpu.CompilerParams(
            dimension_semantics=("parallel",),
        ),
    )(
        time_steps.astype(jnp.int32),
        prev_time_steps.astype(jnp.int32),
        sac,
        somac,
        x3,
        p3,
    )
    return xtp.reshape(B, C, H, W), x0.reshape(B, C, H, W)


# speedup vs baseline: 1.8656x; 1.0579x over previous
import jax
import jax.numpy as jnp
from jax.experimental import pallas as pl
from jax.experimental.pallas import tpu as pltpu

_TAU_NUM_STEPS = 50
_NUM_TRAIN_STEPS = 1000


def _tau_tables():
    betas = jnp.linspace(1e-4, 0.02, _NUM_TRAIN_STEPS, dtype=jnp.float32)
    alphas = 1.0 - betas
    alpha_bars = jnp.cumprod(alphas)
    tau = jnp.linspace(0, _NUM_TRAIN_STEPS - 1, _TAU_NUM_STEPS).astype(jnp.int32)
    tau_alpha_bars = jnp.take(alpha_bars, tau)
    sac = jnp.sqrt(tau_alpha_bars)
    somac = jnp.sqrt(1.0 - tau_alpha_bars)
    return sac, somac


def _body(ts_ref, pts_ref, sac_ref, somac_ref, xt_ref, pn_ref, xtp_ref, x0_ref):
    i = pl.program_id(0)
    g = xt_ref.shape[0]
    for k in range(g):
        s = i * g + k
        t = ts_ref[s]
        p = pts_ref[s]
        sac = sac_ref[t]
        somac = somac_ref[t]
        psac = sac_ref[p]
        psomac = somac_ref[p]
        dc = jnp.sqrt(jnp.maximum(psomac * psomac, jnp.float32(1e-8)))
        rsac = 1.0 / sac
        c = rsac
        d = -rsac * somac
        a = psac * c
        b = psac * d + dc
        xt_v = xt_ref[k]
        pn = pn_ref[k]
        x0_ref[k] = c * xt_v + d * pn
        xtp_ref[k] = a * xt_v + b * pn


_GROUP = 4


def kernel(xt, predicted_noise, time_steps, prev_time_steps):
    B, C, H, W = xt.shape
    rows = C * H * W // W
    sac, somac = _tau_tables()
    x3 = xt.reshape(B, rows, W)
    p3 = predicted_noise.reshape(B, rows, W)
    g = _GROUP
    grid_spec = pltpu.PrefetchScalarGridSpec(
        num_scalar_prefetch=4,
        grid=(B // g,),
        in_specs=[
            pl.BlockSpec((g, rows, W), lambda i, *_: (i, 0, 0)),
            pl.BlockSpec((g, rows, W), lambda i, *_: (i, 0, 0)),
        ],
        out_specs=[
            pl.BlockSpec((g, rows, W), lambda i, *_: (i, 0, 0)),
            pl.BlockSpec((g, rows, W), lambda i, *_: (i, 0, 0)),
        ],
    )
    xtp, x0 = pl.pallas_call(
        _body,
        grid_spec=grid_spec,
        out_shape=[jax.ShapeDtypeStruct((B, rows, W), jnp.float32)] * 2,
        compiler_params=pltpu.CompilerParams(
            dimension_semantics=("parallel",),
        ),
    )(
        time_steps.astype(jnp.int32),
        prev_time_steps.astype(jnp.int32),
        sac,
        somac,
        x3,
        p3,
    )
    return xtp.reshape(B, C, H, W), x0.reshape(B, C, H, W)


# speedup vs baseline: 1.8847x; 1.0102x over previous
import jax
import jax.numpy as jnp
from jax.experimental import pallas as pl
from jax.experimental.pallas import tpu as pltpu

_TAU_NUM_STEPS = 50
_NUM_TRAIN_STEPS = 1000


def _tau_tables():
    betas = jnp.linspace(1e-4, 0.02, _NUM_TRAIN_STEPS, dtype=jnp.float32)
    alphas = 1.0 - betas
    alpha_bars = jnp.cumprod(alphas)
    tau = jnp.linspace(0, _NUM_TRAIN_STEPS - 1, _TAU_NUM_STEPS).astype(jnp.int32)
    tau_alpha_bars = jnp.take(alpha_bars, tau)
    sac = jnp.sqrt(tau_alpha_bars)
    somac = jnp.sqrt(1.0 - tau_alpha_bars)
    return sac, somac


def _body(ts_ref, pts_ref, sac_ref, somac_ref, xt_ref, pn_ref, xtp_ref, x0_ref):
    i = pl.program_id(0)
    g = xt_ref.shape[0]
    for k in range(g):
        s = i * g + k
        t = ts_ref[s]
        p = pts_ref[s]
        sac = sac_ref[t]
        somac = somac_ref[t]
        psac = sac_ref[p]
        psomac = somac_ref[p]
        dc = jnp.sqrt(jnp.maximum(psomac * psomac, jnp.float32(1e-8)))
        rsac = 1.0 / sac
        c = rsac
        d = -rsac * somac
        a = psac * c
        b = psac * d + dc
        xt_v = xt_ref[k]
        pn = pn_ref[k]
        x0_ref[k] = c * xt_v + d * pn
        xtp_ref[k] = a * xt_v + b * pn


_GROUP = 8


def kernel(xt, predicted_noise, time_steps, prev_time_steps):
    B, C, H, W = xt.shape
    rows = C * H * W // W
    sac, somac = _tau_tables()
    x3 = xt.reshape(B, rows, W)
    p3 = predicted_noise.reshape(B, rows, W)
    g = _GROUP
    grid_spec = pltpu.PrefetchScalarGridSpec(
        num_scalar_prefetch=4,
        grid=(B // g,),
        in_specs=[
            pl.BlockSpec((g, rows, W), lambda i, *_: (i, 0, 0)),
            pl.BlockSpec((g, rows, W), lambda i, *_: (i, 0, 0)),
        ],
        out_specs=[
            pl.BlockSpec((g, rows, W), lambda i, *_: (i, 0, 0)),
            pl.BlockSpec((g, rows, W), lambda i, *_: (i, 0, 0)),
        ],
    )
    xtp, x0 = pl.pallas_call(
        _body,
        grid_spec=grid_spec,
        out_shape=[jax.ShapeDtypeStruct((B, rows, W), jnp.float32)] * 2,
        compiler_params=pltpu.CompilerParams(
            dimension_semantics=("parallel",),
        ),
    )(
        time_steps.astype(jnp.int32),
        prev_time_steps.astype(jnp.int32),
        sac,
        somac,
        x3,
        p3,
    )
    return xtp.reshape(B, C, H, W), x0.reshape(B, C, H, W)
